# baseline (speedup 1.0000x reference)
import functools

import jax
import jax.numpy as jnp
import numpy as np
from jax import lax
from jax.experimental import pallas as pl
from jax.experimental.pallas import tpu as pltpu
from jax.experimental.pallas import tpu_sc as plsc

VOCAB = 100000
EMB = 64
HIDDEN = 512
N_CLASSES = 100
BATCH = 4096
PAD_LEN = 200

_NC = 2
_NS = 16
_NW = _NC * _NS
_BPW = BATCH // _NW
_CH0 = 128
_CH1 = PAD_LEN - _CH0
_NBUF = 4

_PERM = np.concatenate(
    [np.arange(g * 32 + p, (g + 1) * 32, 2) for g in range(EMB // 32) for p in (0, 1)]
)


def _gather_mean_body(
    text_hbm, table_hbm, out_hbm, idx_v, rows_v, out_v, sem0, sem1, sem2, sem3
):
    wid = lax.axis_index("s") * _NC + lax.axis_index("c")
    base = wid * _BPW
    pltpu.sync_copy(text_hbm.at[pl.ds(base, _BPW)], idx_v)
    sems = (sem0, sem1, sem2, sem3)

    def _copies(s, b):
        sc = jnp.minimum(s, _BPW - 1)
        c0 = pltpu.make_async_copy(
            table_hbm.at[idx_v.at[sc, pl.ds(0, _CH0)]],
            rows_v.at[b, pl.ds(0, _CH0)],
            sems[b],
        )
        c1 = pltpu.make_async_copy(
            table_hbm.at[idx_v.at[sc, pl.ds(_CH0, _CH1)]],
            rows_v.at[b, pl.ds(_CH0, _CH1)],
            sems[b],
        )
        return c0, c1

    def _issue(s, b):
        for c in _copies(s, b):
            c.start()

    def _drain(s, b):
        for c in _copies(s, b):
            c.wait()

    scale = jnp.float32(1.0 / PAD_LEN)
    zeros = tuple(jnp.zeros((16,), jnp.float32) for _ in range(EMB // 16))
    _UNROLL = 4

    mask_hi = jnp.full((16,), jnp.int32(-65536))

    def _accum(s, b):
        def tokn(t, acc):
            t0 = _UNROLL * t
            for u in range(_UNROLL):
                a = list(acc)
                for g in range(EMB // 32):
                    v = rows_v[b, t0 + u, pl.ds(16 * g, 16)]
                    ev = lax.bitcast_convert_type(
                        lax.shift_left(v, 16), jnp.float32
                    )
                    od = lax.bitcast_convert_type(v & mask_hi, jnp.float32)
                    a[2 * g] = a[2 * g] + ev
                    a[2 * g + 1] = a[2 * g + 1] + od
                acc = tuple(a)
            return acc

        acc = lax.fori_loop(0, PAD_LEN // _UNROLL, tokn, zeros)
        for c in range(EMB // 16):
            out_v[s, pl.ds(16 * c, 16)] = acc[c] * scale

    for b in range(_NBUF):
        _issue(b, b)

    def ring_body(i, carry):
        s0 = _NBUF * i
        for b in range(_NBUF):
            s = s0 + b
            _drain(s, b)
            _accum(s, b)
            _issue(s + _NBUF, b)
        return carry

    lax.fori_loop(0, _BPW // _NBUF, ring_body, 0)
    for b in range(_NBUF):
        _drain(_BPW + b, b)
    pltpu.sync_copy(out_v, out_hbm.at[pl.ds(base, _BPW)])


def _gather_mean(text, table_i32):
    mesh = plsc.VectorSubcoreMesh(
        core_axis_name="c", subcore_axis_name="s", num_cores=_NC, num_subcores=_NS
    )
    k = pl.kernel(
        _gather_mean_body,
        out_type=jax.ShapeDtypeStruct((BATCH, EMB), jnp.float32),
        mesh=mesh,
        compiler_params=pltpu.CompilerParams(use_tc_tiling_on_sc=False),
        scratch_types=[
            pltpu.VMEM((_BPW, PAD_LEN), jnp.int32),
            pltpu.VMEM((_NBUF, PAD_LEN, EMB // 2), jnp.int32),
            pltpu.VMEM((_BPW, EMB), jnp.float32),
            pltpu.SemaphoreType.DMA,
            pltpu.SemaphoreType.DMA,
            pltpu.SemaphoreType.DMA,
            pltpu.SemaphoreType.DMA,
        ],
    )
    return k(text, table_i32)


_NC_PAD = 128


def _mlp_body(avg_ref, w1_ref, b1_ref, w2_ref, b2_ref, out_ref):
    h = (
        jnp.dot(avg_ref[:], w1_ref[:], preferred_element_type=jnp.float32)
        + b1_ref[:]
    )
    out_ref[:] = (
        jnp.dot(h, w2_ref[:], preferred_element_type=jnp.float32) + b2_ref[:]
    )


def _mlp(avg, W1, b1, W2, b2):
    blk = 1024
    W2p = jnp.pad(W2, ((0, 0), (0, _NC_PAD - N_CLASSES)))
    b2p = jnp.pad(b2, (0, _NC_PAD - N_CLASSES))
    out = pl.pallas_call(
        _mlp_body,
        grid=(BATCH // blk,),
        in_specs=[
            pl.BlockSpec((blk, EMB), lambda i: (i, 0)),
            pl.BlockSpec((EMB, HIDDEN), lambda i: (0, 0)),
            pl.BlockSpec((1, HIDDEN), lambda i: (0, 0)),
            pl.BlockSpec((HIDDEN, _NC_PAD), lambda i: (0, 0)),
            pl.BlockSpec((1, _NC_PAD), lambda i: (0, 0)),
        ],
        out_specs=pl.BlockSpec((blk, _NC_PAD), lambda i: (i, 0)),
        out_shape=jax.ShapeDtypeStruct((BATCH, _NC_PAD), jnp.float32),
    )(avg, W1, b1.reshape(1, HIDDEN), W2p, b2p.reshape(1, _NC_PAD))
    return out[:, :N_CLASSES]


def kernel(text, words_per_sentence, table, W1, b1, W2, b2):
    table_i32 = lax.bitcast_convert_type(
        table.astype(jnp.bfloat16).reshape(VOCAB, EMB // 2, 2), jnp.int32
    )
    avg_perm = _gather_mean(text, table_i32)
    W1_perm = W1[_PERM, :]
    return _mlp(avg_perm, W1_perm, b1, W2, b2)


# speedup vs baseline: 1.7343x; 1.7343x over previous
import functools

import jax
import jax.numpy as jnp
import numpy as np
from jax import lax
from jax.experimental import pallas as pl
from jax.experimental.pallas import tpu as pltpu
from jax.experimental.pallas import tpu_sc as plsc

VOCAB = 100000
EMB = 64
HIDDEN = 512
N_CLASSES = 100
BATCH = 4096
PAD_LEN = 200

_NC = 2
_NS = 16
_NW = _NC * _NS
_BPW = BATCH // _NW
_CH0 = 128
_CH1 = PAD_LEN - _CH0
_NBUF = 4


def _gather_mean_body(
    text_hbm, table_hbm, out_hbm, idx_v, rows_v, out_v, sem0, sem1, sem2, sem3
):
    wid = lax.axis_index("s") * _NC + lax.axis_index("c")
    base = wid * _BPW
    pltpu.sync_copy(text_hbm.at[pl.ds(base, _BPW)], idx_v)
    sems = (sem0, sem1, sem2, sem3)

    def _copies(s, b):
        sc = jnp.minimum(s, _BPW - 1)
        c0 = pltpu.make_async_copy(
            table_hbm.at[idx_v.at[sc, pl.ds(0, _CH0)]],
            rows_v.at[b, pl.ds(0, _CH0)],
            sems[b],
        )
        c1 = pltpu.make_async_copy(
            table_hbm.at[idx_v.at[sc, pl.ds(_CH0, _CH1)]],
            rows_v.at[b, pl.ds(_CH0, _CH1)],
            sems[b],
        )
        return c0, c1

    def _issue(s, b):
        for c in _copies(s, b):
            c.start()

    def _drain(s, b):
        for c in _copies(s, b):
            c.wait()

    scale = jnp.float32(1.0 / PAD_LEN)
    zeros = tuple(jnp.zeros((16,), jnp.float32) for _ in range(EMB // 16))
    _UNROLL = 4

    mask_hi = jnp.full((16,), jnp.int32(-65536))

    def _accum(s, b):
        def tokn(t, acc):
            t0 = _UNROLL * t
            for u in range(_UNROLL):
                a = list(acc)
                for g in range(EMB // 32):
                    v = rows_v[b, t0 + u, pl.ds(16 * g, 16)]
                    lo = lax.bitcast_convert_type(
                        lax.shift_left(v, 16), jnp.float32
                    )
                    hi = lax.bitcast_convert_type(v & mask_hi, jnp.float32)
                    a[g] = a[g] + lo
                    a[g + 2] = a[g + 2] + hi
                acc = tuple(a)
            return acc

        acc = lax.fori_loop(0, PAD_LEN // _UNROLL, tokn, zeros)
        for c in range(EMB // 16):
            out_v[s, pl.ds(16 * c, 16)] = acc[c] * scale

    for b in range(_NBUF):
        _issue(b, b)

    def ring_body(i, carry):
        s0 = _NBUF * i
        for b in range(_NBUF):
            s = s0 + b
            _drain(s, b)
            _accum(s, b)
            _issue(s + _NBUF, b)
        return carry

    lax.fori_loop(0, _BPW // _NBUF, ring_body, 0)
    for b in range(_NBUF):
        _drain(_BPW + b, b)
    pltpu.sync_copy(out_v, out_hbm.at[pl.ds(base, _BPW)])


def _gather_mean(text, table_i32):
    mesh = plsc.VectorSubcoreMesh(
        core_axis_name="c", subcore_axis_name="s", num_cores=_NC, num_subcores=_NS
    )
    k = pl.kernel(
        _gather_mean_body,
        out_type=jax.ShapeDtypeStruct((BATCH, EMB), jnp.float32),
        mesh=mesh,
        compiler_params=pltpu.CompilerParams(use_tc_tiling_on_sc=False),
        scratch_types=[
            pltpu.VMEM((_BPW, PAD_LEN), jnp.int32),
            pltpu.VMEM((_NBUF, PAD_LEN, EMB // 2), jnp.int32),
            pltpu.VMEM((_BPW, EMB), jnp.float32),
            pltpu.SemaphoreType.DMA,
            pltpu.SemaphoreType.DMA,
            pltpu.SemaphoreType.DMA,
            pltpu.SemaphoreType.DMA,
        ],
    )
    return k(text, table_i32)


_NC_PAD = 128


def _mlp_body(avg_ref, w1_ref, b1_ref, w2_ref, b2_ref, out_ref):
    h = (
        jnp.dot(avg_ref[:], w1_ref[:], preferred_element_type=jnp.float32)
        + b1_ref[:]
    )
    out_ref[:] = (
        jnp.dot(h, w2_ref[:], preferred_element_type=jnp.float32) + b2_ref[:]
    )


def _mlp(avg, W1, b1, W2, b2):
    blk = 1024
    W2p = jnp.pad(W2, ((0, 0), (0, _NC_PAD - N_CLASSES)))
    b2p = jnp.pad(b2, (0, _NC_PAD - N_CLASSES))
    out = pl.pallas_call(
        _mlp_body,
        grid=(BATCH // blk,),
        in_specs=[
            pl.BlockSpec((blk, EMB), lambda i: (i, 0)),
            pl.BlockSpec((EMB, HIDDEN), lambda i: (0, 0)),
            pl.BlockSpec((1, HIDDEN), lambda i: (0, 0)),
            pl.BlockSpec((HIDDEN, _NC_PAD), lambda i: (0, 0)),
            pl.BlockSpec((1, _NC_PAD), lambda i: (0, 0)),
        ],
        out_specs=pl.BlockSpec((blk, _NC_PAD), lambda i: (i, 0)),
        out_shape=jax.ShapeDtypeStruct((BATCH, _NC_PAD), jnp.float32),
    )(avg, W1, b1.reshape(1, HIDDEN), W2p, b2p.reshape(1, _NC_PAD))
    return out[:, :N_CLASSES]


def _pack_body(x_ref, o_ref):
    bits = lax.bitcast_convert_type(x_ref[:], jnp.int32)
    lsb = lax.shift_right_logical(bits, 16) & jnp.int32(1)
    rnd = bits + jnp.int32(32767) + lsb
    lo = lax.shift_right_logical(rnd[:, : EMB // 2], 16)
    hi = rnd[:, EMB // 2 :] & jnp.int32(-65536)
    o_ref[:] = lo | hi


def _pack_table(table):
    blk = 10000
    return pl.pallas_call(
        _pack_body,
        grid=(VOCAB // blk,),
        in_specs=[pl.BlockSpec((blk, EMB), lambda i: (i, 0))],
        out_specs=pl.BlockSpec((blk, EMB // 2), lambda i: (i, 0)),
        out_shape=jax.ShapeDtypeStruct((VOCAB, EMB // 2), jnp.int32),
    )(table)


def kernel(text, words_per_sentence, table, W1, b1, W2, b2):
    avg = _gather_mean(text, _pack_table(table))
    return _mlp(avg, W1, b1, W2, b2)


# speedup vs baseline: 2.0124x; 1.1604x over previous
import functools

import jax
import jax.numpy as jnp
import numpy as np
from jax import lax
from jax.experimental import pallas as pl
from jax.experimental.pallas import tpu as pltpu
from jax.experimental.pallas import tpu_sc as plsc

VOCAB = 100000
EMB = 64
HIDDEN = 512
N_CLASSES = 100
BATCH = 4096
PAD_LEN = 200

_NC = 2
_NS = 16
_NW = _NC * _NS
_BPW = BATCH // _NW
_CH0 = 128
_CH1 = PAD_LEN - _CH0
_NBUF = 4


def _gather_mean_body(
    text_hbm, table_hbm, out_hbm, idx_v, rows_v, out_v, sem0, sem1, sem2, sem3
):
    wid = lax.axis_index("s") * _NC + lax.axis_index("c")
    base = wid * _BPW
    pltpu.sync_copy(text_hbm.at[pl.ds(base, _BPW)], idx_v)
    sems = (sem0, sem1, sem2, sem3)

    def _copies(s, b):
        sc = jnp.minimum(s, _BPW - 1)
        c0 = pltpu.make_async_copy(
            table_hbm.at[idx_v.at[sc, pl.ds(0, _CH0)]],
            rows_v.at[b, pl.ds(0, _CH0)],
            sems[b],
        )
        c1 = pltpu.make_async_copy(
            table_hbm.at[idx_v.at[sc, pl.ds(_CH0, _CH1)]],
            rows_v.at[b, pl.ds(_CH0, _CH1)],
            sems[b],
        )
        return c0, c1

    def _issue(s, b):
        for c in _copies(s, b):
            c.start()

    def _drain(s, b):
        for c in _copies(s, b):
            c.wait()

    scale = jnp.float32(1.0 / PAD_LEN)
    zeros = tuple(jnp.zeros((16,), jnp.float32) for _ in range(EMB // 16))
    _UNROLL = 4

    mask_hi = jnp.full((16,), jnp.int32(-65536))

    def _accum(s, b):
        def tokn(t, acc):
            t0 = _UNROLL * t
            for u in range(_UNROLL):
                a = list(acc)
                for g in range(EMB // 32):
                    v = rows_v[b, t0 + u, pl.ds(16 * g, 16)]
                    lo = lax.bitcast_convert_type(
                        lax.shift_left(v, 16), jnp.float32
                    )
                    hi = lax.bitcast_convert_type(v & mask_hi, jnp.float32)
                    a[g] = a[g] + lo
                    a[g + 2] = a[g + 2] + hi
                acc = tuple(a)
            return acc

        acc = lax.fori_loop(0, PAD_LEN // _UNROLL, tokn, zeros)
        for c in range(EMB // 16):
            out_v[s, pl.ds(16 * c, 16)] = acc[c] * scale

    for b in range(_NBUF):
        _issue(b, b)

    def ring_body(i, carry):
        s0 = _NBUF * i
        for b in range(_NBUF):
            s = s0 + b
            _drain(s, b)
            _accum(s, b)
            _issue(s + _NBUF, b)
        return carry

    lax.fori_loop(0, _BPW // _NBUF, ring_body, 0)
    for b in range(_NBUF):
        _drain(_BPW + b, b)
    pltpu.sync_copy(out_v, out_hbm.at[pl.ds(base, _BPW)])


def _gather_mean(text, table_i32):
    mesh = plsc.VectorSubcoreMesh(
        core_axis_name="c", subcore_axis_name="s", num_cores=_NC, num_subcores=_NS
    )
    k = pl.kernel(
        _gather_mean_body,
        out_type=jax.ShapeDtypeStruct((BATCH, EMB), jnp.float32),
        mesh=mesh,
        compiler_params=pltpu.CompilerParams(use_tc_tiling_on_sc=False),
        scratch_types=[
            pltpu.VMEM((_BPW, PAD_LEN), jnp.int32),
            pltpu.VMEM((_NBUF, PAD_LEN, EMB // 2), jnp.int32),
            pltpu.VMEM((_BPW, EMB), jnp.float32),
            pltpu.SemaphoreType.DMA,
            pltpu.SemaphoreType.DMA,
            pltpu.SemaphoreType.DMA,
            pltpu.SemaphoreType.DMA,
        ],
    )
    return k(text, table_i32)


_NC_PAD = 128


def _mlp_body(avg_ref, w1_ref, b1_ref, w2_ref, b2_ref, out_ref):
    h = (
        jnp.dot(avg_ref[:], w1_ref[:], preferred_element_type=jnp.float32)
        + b1_ref[:]
    )
    out_ref[:] = (
        jnp.dot(h, w2_ref[:], preferred_element_type=jnp.float32) + b2_ref[:]
    )


def _mlp(avg, W1, b1, W2, b2):
    blk = 1024
    W2p = jnp.pad(W2, ((0, 0), (0, _NC_PAD - N_CLASSES)))
    b2p = jnp.pad(b2, (0, _NC_PAD - N_CLASSES))
    out = pl.pallas_call(
        _mlp_body,
        grid=(BATCH // blk,),
        in_specs=[
            pl.BlockSpec((blk, EMB), lambda i: (i, 0)),
            pl.BlockSpec((EMB, HIDDEN), lambda i: (0, 0)),
            pl.BlockSpec((1, HIDDEN), lambda i: (0, 0)),
            pl.BlockSpec((HIDDEN, _NC_PAD), lambda i: (0, 0)),
            pl.BlockSpec((1, _NC_PAD), lambda i: (0, 0)),
        ],
        out_specs=pl.BlockSpec((blk, _NC_PAD), lambda i: (i, 0)),
        out_shape=jax.ShapeDtypeStruct((BATCH, _NC_PAD), jnp.float32),
    )(avg, W1, b1.reshape(1, HIDDEN), W2p, b2p.reshape(1, _NC_PAD))
    return out[:, :N_CLASSES]


_FBLK = 8


def _pack_body(lo_ref, hi_ref, o_ref):
    def rnd(x):
        bits = lax.bitcast_convert_type(x, jnp.int32)
        lsb = lax.shift_right_logical(bits, 16) & jnp.int32(1)
        return bits + jnp.int32(32767) + lsb

    lo = lax.shift_right_logical(rnd(lo_ref[:]), 16)
    hi = rnd(hi_ref[:]) & jnp.int32(-65536)
    o_ref[:] = lo | hi


def _pack_table(table):
    tt = table.T
    packed_w = pl.pallas_call(
        _pack_body,
        grid=((EMB // 2) // _FBLK,),
        in_specs=[
            pl.BlockSpec((_FBLK, VOCAB), lambda g: (g, 0)),
            pl.BlockSpec((_FBLK, VOCAB), lambda g: (g + (EMB // 2) // _FBLK, 0)),
        ],
        out_specs=pl.BlockSpec((_FBLK, VOCAB), lambda g: (g, 0)),
        out_shape=jax.ShapeDtypeStruct((EMB // 2, VOCAB), jnp.int32),
    )(tt, tt)
    return packed_w.T


def kernel(text, words_per_sentence, table, W1, b1, W2, b2):
    avg = _gather_mean(text, _pack_table(table))
    return _mlp(avg, W1, b1, W2, b2)


# speedup vs baseline: 2.0237x; 1.0056x over previous
import functools

import jax
import jax.numpy as jnp
import numpy as np
from jax import lax
from jax.experimental import pallas as pl
from jax.experimental.pallas import tpu as pltpu
from jax.experimental.pallas import tpu_sc as plsc

VOCAB = 100000
EMB = 64
HIDDEN = 512
N_CLASSES = 100
BATCH = 4096
PAD_LEN = 200

_NC = 2
_NS = 16
_NW = _NC * _NS
_BPW = BATCH // _NW
_CH0 = 128
_CH1 = PAD_LEN - _CH0
_NBUF = 8


def _gather_mean_body(
    text_hbm, table_hbm, out_hbm, idx_v, rows_v, out_v, sem0, sem1, sem2, sem3, sem4, sem5, sem6, sem7
):
    wid = lax.axis_index("s") * _NC + lax.axis_index("c")
    base = wid * _BPW
    pltpu.sync_copy(text_hbm.at[pl.ds(base, _BPW)], idx_v)
    sems = (sem0, sem1, sem2, sem3, sem4, sem5, sem6, sem7)

    def _copies(s, b):
        sc = jnp.minimum(s, _BPW - 1)
        c0 = pltpu.make_async_copy(
            table_hbm.at[idx_v.at[sc, pl.ds(0, _CH0)]],
            rows_v.at[b, pl.ds(0, _CH0)],
            sems[b],
        )
        c1 = pltpu.make_async_copy(
            table_hbm.at[idx_v.at[sc, pl.ds(_CH0, _CH1)]],
            rows_v.at[b, pl.ds(_CH0, _CH1)],
            sems[b],
        )
        return c0, c1

    def _issue(s, b):
        for c in _copies(s, b):
            c.start()

    def _drain(s, b):
        for c in _copies(s, b):
            c.wait()

    scale = jnp.float32(1.0 / PAD_LEN)
    zeros = tuple(jnp.zeros((16,), jnp.float32) for _ in range(EMB // 16))
    _UNROLL = 4

    mask_hi = jnp.full((16,), jnp.int32(-65536))

    def _accum(s, b):
        def tokn(t, acc):
            t0 = _UNROLL * t
            for u in range(_UNROLL):
                a = list(acc)
                for g in range(EMB // 32):
                    v = rows_v[b, t0 + u, pl.ds(16 * g, 16)]
                    lo = lax.bitcast_convert_type(
                        lax.shift_left(v, 16), jnp.float32
                    )
                    hi = lax.bitcast_convert_type(v & mask_hi, jnp.float32)
                    a[g] = a[g] + lo
                    a[g + 2] = a[g + 2] + hi
                acc = tuple(a)
            return acc

        acc = lax.fori_loop(0, PAD_LEN // _UNROLL, tokn, zeros)
        for c in range(EMB // 16):
            out_v[s, pl.ds(16 * c, 16)] = acc[c] * scale

    for b in range(_NBUF):
        _issue(b, b)

    def ring_body(i, carry):
        s0 = _NBUF * i
        for b in range(_NBUF):
            s = s0 + b
            _drain(s, b)
            _accum(s, b)
            _issue(s + _NBUF, b)
        return carry

    lax.fori_loop(0, _BPW // _NBUF, ring_body, 0)
    for b in range(_NBUF):
        _drain(_BPW + b, b)
    pltpu.sync_copy(out_v, out_hbm.at[pl.ds(base, _BPW)])


def _gather_mean(text, table_i32):
    mesh = plsc.VectorSubcoreMesh(
        core_axis_name="c", subcore_axis_name="s", num_cores=_NC, num_subcores=_NS
    )
    k = pl.kernel(
        _gather_mean_body,
        out_type=jax.ShapeDtypeStruct((BATCH, EMB), jnp.float32),
        mesh=mesh,
        compiler_params=pltpu.CompilerParams(use_tc_tiling_on_sc=False),
        scratch_types=[
            pltpu.VMEM((_BPW, PAD_LEN), jnp.int32),
            pltpu.VMEM((_NBUF, PAD_LEN, EMB // 2), jnp.int32),
            pltpu.VMEM((_BPW, EMB), jnp.float32),
            pltpu.SemaphoreType.DMA,
            pltpu.SemaphoreType.DMA,
            pltpu.SemaphoreType.DMA,
            pltpu.SemaphoreType.DMA,
            pltpu.SemaphoreType.DMA,
            pltpu.SemaphoreType.DMA,
            pltpu.SemaphoreType.DMA,
            pltpu.SemaphoreType.DMA,
        ],
    )
    return k(text, table_i32)


_NC_PAD = 128


def _mlp_body(avg_ref, w1_ref, b1_ref, w2_ref, b2_ref, out_ref):
    h = (
        jnp.dot(avg_ref[:], w1_ref[:], preferred_element_type=jnp.float32)
        + b1_ref[:]
    )
    out_ref[:] = (
        jnp.dot(h, w2_ref[:], preferred_element_type=jnp.float32) + b2_ref[:]
    )


def _mlp(avg, W1, b1, W2, b2):
    blk = 1024
    W2p = jnp.pad(W2, ((0, 0), (0, _NC_PAD - N_CLASSES)))
    b2p = jnp.pad(b2, (0, _NC_PAD - N_CLASSES))
    out = pl.pallas_call(
        _mlp_body,
        grid=(BATCH // blk,),
        in_specs=[
            pl.BlockSpec((blk, EMB), lambda i: (i, 0)),
            pl.BlockSpec((EMB, HIDDEN), lambda i: (0, 0)),
            pl.BlockSpec((1, HIDDEN), lambda i: (0, 0)),
            pl.BlockSpec((HIDDEN, _NC_PAD), lambda i: (0, 0)),
            pl.BlockSpec((1, _NC_PAD), lambda i: (0, 0)),
        ],
        out_specs=pl.BlockSpec((blk, _NC_PAD), lambda i: (i, 0)),
        out_shape=jax.ShapeDtypeStruct((BATCH, _NC_PAD), jnp.float32),
    )(avg, W1, b1.reshape(1, HIDDEN), W2p, b2p.reshape(1, _NC_PAD))
    return out[:, :N_CLASSES]


_FBLK = 8


def _pack_body(lo_ref, hi_ref, o_ref):
    def rnd(x):
        bits = lax.bitcast_convert_type(x, jnp.int32)
        lsb = lax.shift_right_logical(bits, 16) & jnp.int32(1)
        return bits + jnp.int32(32767) + lsb

    lo = lax.shift_right_logical(rnd(lo_ref[:]), 16)
    hi = rnd(hi_ref[:]) & jnp.int32(-65536)
    o_ref[:] = lo | hi


def _pack_table(table):
    tt = table.T
    packed_w = pl.pallas_call(
        _pack_body,
        grid=((EMB // 2) // _FBLK,),
        in_specs=[
            pl.BlockSpec((_FBLK, VOCAB), lambda g: (g, 0)),
            pl.BlockSpec((_FBLK, VOCAB), lambda g: (g + (EMB // 2) // _FBLK, 0)),
        ],
        out_specs=pl.BlockSpec((_FBLK, VOCAB), lambda g: (g, 0)),
        out_shape=jax.ShapeDtypeStruct((EMB // 2, VOCAB), jnp.int32),
    )(tt, tt)
    return packed_w.T


def kernel(text, words_per_sentence, table, W1, b1, W2, b2):
    avg = _gather_mean(text, _pack_table(table))
    return _mlp(avg, W1, b1, W2, b2)
